# Initial kernel scaffold; baseline (speedup 1.0000x reference)
import jax
import jax.numpy as jnp
from jax.experimental import pallas as pl


def kernel(x, table):
    raise NotImplementedError("write your pallas kernel here")


# speedup vs baseline: 1.4693x; 1.4693x over previous
import functools

import jax
import jax.numpy as jnp
from jax import lax
from jax.experimental import pallas as pl
from jax.experimental.pallas import tpu as pltpu
from jax.experimental.pallas import tpu_sc as plsc

NC = 2
NS = 16
NW = NC * NS

CHUNK = 1280


@functools.lru_cache(maxsize=None)
def _make_gather(n_rows: int, dim: int, vocab: int):
    assert n_rows % NW == 0
    per_w = n_rows // NW
    assert per_w % CHUNK == 0
    n_chunks = per_w // CHUNK

    mesh = plsc.VectorSubcoreMesh(core_axis_name="c", subcore_axis_name="s")

    @functools.partial(
        pl.kernel,
        out_type=jax.ShapeDtypeStruct((n_rows, dim), jnp.float32),
        mesh=mesh,
        scratch_types=[
            pltpu.VMEM((CHUNK,), jnp.int32),
            pltpu.VMEM((CHUNK, dim), jnp.float32),
            pltpu.SemaphoreType.DMA,
        ],
        compiler_params=pltpu.CompilerParams(use_tc_tiling_on_sc=False),
    )
    def gather_kernel(x_hbm, table_hbm, out_hbm, idx_v, rows_v, sem):
        wid = lax.axis_index("s") * NC + lax.axis_index("c")
        base = wid * per_w

        def chunk_body(i, carry):
            off = base + i * CHUNK
            pltpu.sync_copy(x_hbm.at[pl.ds(off, CHUNK)], idx_v)
            pltpu.async_copy(table_hbm.at[idx_v], rows_v, sem).wait()
            pltpu.sync_copy(rows_v, out_hbm.at[pl.ds(off, CHUNK)])
            return carry

        lax.fori_loop(0, n_chunks, chunk_body, 0)

    return gather_kernel


def kernel(x, table):
    b, l = x.shape
    vocab, dim = table.shape
    flat_idx = x.reshape(b * l)
    out = _make_gather(b * l, dim, vocab)(flat_idx, table)
    return out.reshape(b, l, dim)


# speedup vs baseline: 1.4930x; 1.0161x over previous
import functools

import jax
import jax.numpy as jnp
from jax import lax
from jax.experimental import pallas as pl
from jax.experimental.pallas import tpu as pltpu
from jax.experimental.pallas import tpu_sc as plsc

NC = 2
NS = 16
NW = NC * NS

CHUNK = 1280


@functools.lru_cache(maxsize=None)
def _make_gather(n_rows: int, dim: int, vocab: int):
    assert n_rows % NW == 0
    per_w = n_rows // NW
    assert per_w % CHUNK == 0
    n_chunks = per_w // CHUNK
    assert n_chunks >= 2

    mesh = plsc.VectorSubcoreMesh(core_axis_name="c", subcore_axis_name="s")

    @functools.partial(
        pl.kernel,
        out_type=jax.ShapeDtypeStruct((n_rows, dim), jnp.float32),
        mesh=mesh,
        scratch_types=[
            pltpu.VMEM((CHUNK,), jnp.int32),
            pltpu.VMEM((CHUNK,), jnp.int32),
            pltpu.VMEM((CHUNK, dim), jnp.float32),
            pltpu.VMEM((CHUNK, dim), jnp.float32),
            pltpu.SemaphoreType.DMA,
            pltpu.SemaphoreType.DMA,
            pltpu.SemaphoreType.DMA,
            pltpu.SemaphoreType.DMA,
            pltpu.SemaphoreType.DMA,
            pltpu.SemaphoreType.DMA,
        ],
        compiler_params=pltpu.CompilerParams(use_tc_tiling_on_sc=False),
    )
    def gather_kernel(x_hbm, table_hbm, out_hbm,
                      idx0, idx1, rows0, rows1,
                      isem0, isem1, gsem0, gsem1, osem0, osem1):
        wid = lax.axis_index("s") * NC + lax.axis_index("c")
        base = wid * per_w
        idx_b = (idx0, idx1)
        rows_b = (rows0, rows1)
        isem = (isem0, isem1)
        gsem = (gsem0, gsem1)
        osem = (osem0, osem1)

        def issue_idx(i, b):
            off = base + i * CHUNK
            return pltpu.async_copy(x_hbm.at[pl.ds(off, CHUNK)], idx_b[b], isem[b])

        def issue_gather(b):
            return pltpu.async_copy(table_hbm.at[idx_b[b]], rows_b[b], gsem[b])

        def issue_store(i, b):
            off = base + i * CHUNK
            return pltpu.async_copy(rows_b[b], out_hbm.at[pl.ds(off, CHUNK)], osem[b])

        hA = [None] * n_chunks
        hB = [None] * n_chunks
        hC = [None] * n_chunks

        hA[0] = issue_idx(0, 0)
        hA[1] = issue_idx(1, 1)
        hA[0].wait()
        hB[0] = issue_gather(0)
        for i in range(n_chunks):
            b = i % 2
            hB[i].wait()
            hC[i] = issue_store(i, b)
            if i + 2 < n_chunks:
                hA[i + 2] = issue_idx(i + 2, b)
            if i + 1 < n_chunks:
                if i >= 1:
                    hC[i - 1].wait()
                hA[i + 1].wait()
                hB[i + 1] = issue_gather(1 - b)
        hC[n_chunks - 2].wait()
        hC[n_chunks - 1].wait()

    return gather_kernel


def kernel(x, table):
    b, l = x.shape
    vocab, dim = table.shape
    flat_idx = x.reshape(b * l)
    out = _make_gather(b * l, dim, vocab)(flat_idx, table)
    return out.reshape(b, l, dim)
